# baseline (speedup 1.0000x reference)
import jax
import jax.numpy as jnp
from jax import lax
from jax.experimental import pallas as pl
from jax.experimental.pallas import tpu as pltpu
from jax.experimental.pallas import tpu_sc as plsc

N_ROWS = 32768
N_CH = 256
N_SEG = 16
N_CORES = 2
N_SUBCORES = 16
LANES = 16

CH_HALF = N_CH // N_CORES
ROWS_PER_TILE = N_ROWS // N_SUBCORES
CHUNK = 128
N_CHUNKS = ROWS_PER_TILE // CHUNK
RING = 4
N_VECS = CH_HALF // LANES


def _rescale_body(feat_hbm, seg2d_hbm, noise_hbm, out_hbm,
                  ring, idx2d, idxs8, pooled_big, pooled, noise_v, inv_v, acc,
                  in0, in1, in2, in3, ot0, ot1, ot2, ot3):
    in_sems = (in0, in1, in2, in3)
    out_sems = (ot0, ot1, ot2, ot3)
    c = lax.axis_index("c")
    s = lax.axis_index("s")
    ch0 = c * CH_HALF
    row0 = s * ROWS_PER_TILE

    def feat_src(k):
        return feat_hbm.at[pl.ds(row0 + k * CHUNK, CHUNK), pl.ds(ch0, CH_HALF)]

    def out_dst(k):
        return out_hbm.at[pl.ds(row0 + k * CHUNK, CHUNK), pl.ds(ch0, CH_HALF)]

    def slot(j):
        return ring.at[pl.ds(j * CHUNK, CHUNK)]

    def start_in(k, j):
        pltpu.async_copy(feat_src(k), slot(j), in_sems[j])

    def wait_in(j):
        pltpu.make_async_copy(feat_src(0), slot(j), in_sems[j]).wait()

    def start_out(k, j):
        pltpu.async_copy(slot(j), out_dst(k), out_sems[j])

    def wait_out(j):
        pltpu.make_async_copy(slot(j), out_dst(0), out_sems[j]).wait()

    def segvec_of(k, g):
        gi = k * CHUNK + g * LANES
        return idx2d[lax.shift_right_logical(gi, 7),
                     pl.ds(lax.bitwise_and(gi, 127), LANES)]

    start_in(0, 0)
    start_in(1, 1)

    pltpu.sync_copy(
        seg2d_hbm.at[pl.ds(s * (ROWS_PER_TILE // CHUNK), ROWS_PER_TILE // CHUNK)],
        idx2d)

    def sbody(r, carry):
        for v in range(CHUNK // LANES):
            idxs8[r, pl.ds(v * LANES, LANES)] = (
                idx2d[r, pl.ds(v * LANES, LANES)] * 8)
        return carry
    lax.fori_loop(0, ROWS_PER_TILE // CHUNK, sbody, 0)

    @pl.when(s == 0)
    def _():
        def zbody(seg, carry):
            for v in range(N_VECS):
                pooled[seg, pl.ds(v * LANES, LANES)] = jnp.zeros(
                    (LANES,), jnp.float32)
            return carry
        lax.fori_loop(0, N_SEG, zbody, 0)

        def z2body(b, carry):
            pltpu.sync_copy(pooled, acc.at[pl.ds(b * N_SEG, N_SEG)])
            return carry
        lax.fori_loop(0, 8, z2body, 0)
    plsc.subcore_barrier()

    def p1body(q, carry):
        for j in range(RING):
            k = q * RING + j
            wait_in(j)
            nj = (j + 2) % RING
            if j < 2:
                start_in(k + 2, nj)
            else:
                @pl.when(q < (N_CHUNKS // RING) - 1)
                def _():
                    start_in(k + 2, nj)
            pltpu.sync_copy(slot(j), acc.at[idxs8.at[k]], add=True)
        return carry
    lax.fori_loop(0, N_CHUNKS // RING, p1body, 0)
    start_in(0, 0)
    start_in(1, 1)
    plsc.subcore_barrier()

    pltpu.sync_copy(acc, pooled_big)
    pltpu.sync_copy(noise_hbm.at[pl.ds(0, N_SEG), pl.ds(ch0, CH_HALF)], noise_v)

    def p2body(seg, carry):
        for v in range(N_VECS):
            p = pooled_big[seg * 8, pl.ds(v * LANES, LANES)]
            nz = noise_v[seg, pl.ds(v * LANES, LANES)]
            inv_v[seg, pl.ds(v * LANES, LANES)] = 1.0 / ((0.875 + 0.25 * nz) * p)
        return carry
    lax.fori_loop(0, N_SEG, p2body, 0)

    def compute_chunk(k, j):
        base_j = j * CHUNK

        def gbody(g, gcarry):
            segvec = segvec_of(k, g)
            s0 = segvec[0]
            base = base_j + g * LANES

            def fast():
                ivs = [inv_v[s0, pl.ds(v * LANES, LANES)]
                       for v in range(N_VECS)]

                @plsc.parallel_loop(0, LANES, unroll=2)
                def mbody(r):
                    for v in range(N_VECS):
                        col = v * LANES
                        ring[base + r, pl.ds(col, LANES)] = (
                            ring[base + r, pl.ds(col, LANES)] * ivs[v])

            def slow():
                for i in range(LANES):
                    seg = segvec[i]

                    def vbody(v, vcarry, _i=i, _seg=seg):
                        col = v * LANES
                        ring[base + _i, pl.ds(col, LANES)] = (
                            ring[base + _i, pl.ds(col, LANES)]
                            * inv_v[_seg, pl.ds(col, LANES)])
                        return vcarry
                    lax.fori_loop(0, N_VECS, vbody, 0)

            lax.cond(s0 == segvec[LANES - 1], fast, slow)
            return gcarry
        lax.fori_loop(0, CHUNK // LANES, gbody, 0)

    def p3body(q, carry):
        for j in range(RING):
            k = q * RING + j
            wait_in(j)
            compute_chunk(k, j)
            start_out(k, j)
            nj = (j + 2) % RING
            if j < 2:
                @pl.when(q > 0)
                def _():
                    wait_out(nj)
                start_in(k + 2, nj)
            else:
                @pl.when(q < (N_CHUNKS // RING) - 1)
                def _():
                    wait_out(nj)
                    start_in(k + 2, nj)
        return carry
    lax.fori_loop(0, N_CHUNKS // RING, p3body, 0)
    wait_out(0)
    wait_out(1)
    wait_out(2)
    wait_out(3)


def kernel(features, segment_ids, rand_noise):
    seg2d = segment_ids.astype(jnp.int32).reshape(N_ROWS // CHUNK, CHUNK)
    mesh = plsc.VectorSubcoreMesh(core_axis_name="c", subcore_axis_name="s")
    run = pl.kernel(
        _rescale_body,
        mesh=mesh,
        out_type=jax.ShapeDtypeStruct((N_ROWS, N_CH), jnp.float32),
        scratch_types=[
            pltpu.VMEM((RING * CHUNK, CH_HALF), jnp.float32),
            pltpu.VMEM((ROWS_PER_TILE // CHUNK, CHUNK), jnp.int32),
            pltpu.VMEM((ROWS_PER_TILE // CHUNK, CHUNK), jnp.int32),
            pltpu.VMEM((8 * N_SEG, CH_HALF), jnp.float32),
            pltpu.VMEM((N_SEG, CH_HALF), jnp.float32),
            pltpu.VMEM((N_SEG, CH_HALF), jnp.float32),
            pltpu.VMEM((N_SEG, CH_HALF), jnp.float32),
            pltpu.VMEM_SHARED((8 * N_SEG, CH_HALF), jnp.float32),
        ] + [pltpu.SemaphoreType.DMA] * 8,
    )
    return run(features, seg2d, rand_noise)


# speedup vs baseline: 1.2394x; 1.2394x over previous
import jax
import jax.numpy as jnp
from jax import lax
from jax.experimental import pallas as pl
from jax.experimental.pallas import tpu as pltpu
from jax.experimental.pallas import tpu_sc as plsc

N_ROWS = 32768
N_CH = 256
N_SEG = 16
N_CORES = 2
N_SUBCORES = 16
LANES = 16

CH_HALF = N_CH // N_CORES
ROWS_PER_TILE = N_ROWS // N_SUBCORES
CHUNK = 128
N_CHUNKS = ROWS_PER_TILE // CHUNK
RING = 4
N_VECS = CH_HALF // LANES


def _rescale_body(feat_hbm, seg2d_hbm, noise_hbm, out_hbm,
                  ring, obuf, idx2d, pooled, noise_v, inv_v, acc,
                  in0, in1, in2, in3, ot0, ot1, ot2, ot3):
    in_sems = (in0, in1, in2, in3)
    out_sems = (ot0, ot1, ot2, ot3)
    c = lax.axis_index("c")
    s = lax.axis_index("s")
    ch0 = c * CH_HALF
    row0 = s * ROWS_PER_TILE

    def feat_src(k):
        return feat_hbm.at[pl.ds(row0 + k * CHUNK, CHUNK), pl.ds(ch0, CH_HALF)]

    def out_dst(k):
        return out_hbm.at[pl.ds(row0 + k * CHUNK, CHUNK), pl.ds(ch0, CH_HALF)]

    def slot(j):
        return ring.at[pl.ds(j * CHUNK, CHUNK)]

    def start_in(k, j):
        pltpu.async_copy(feat_src(k), slot(j), in_sems[j])

    def wait_in(j):
        pltpu.make_async_copy(feat_src(0), slot(j), in_sems[j]).wait()

    def oslot(o):
        return obuf.at[pl.ds(o * CHUNK, CHUNK)]

    def start_out(k, o):
        pltpu.async_copy(oslot(o), out_dst(k), out_sems[o])

    def wait_out(o):
        pltpu.make_async_copy(oslot(o), out_dst(0), out_sems[o]).wait()

    def segvec_of(k, g):
        gi = k * CHUNK + g * LANES
        return idx2d[lax.shift_right_logical(gi, 7),
                     pl.ds(lax.bitwise_and(gi, 127), LANES)]

    start_in(0, 0)
    start_in(1, 1)

    pltpu.sync_copy(
        seg2d_hbm.at[pl.ds(s * (ROWS_PER_TILE // CHUNK), ROWS_PER_TILE // CHUNK)],
        idx2d)

    @pl.when(s == 0)
    def _():
        def zbody(seg, carry):
            for v in range(N_VECS):
                pooled[seg, pl.ds(v * LANES, LANES)] = jnp.zeros(
                    (LANES,), jnp.float32)
            return carry
        lax.fori_loop(0, N_SEG, zbody, 0)
        pltpu.sync_copy(pooled, acc)
    plsc.subcore_barrier()

    def p1body(q, carry):
        for j in range(RING):
            k = q * RING + j
            wait_in(j)
            nj = (j + 2) % RING
            if j < 2:
                start_in(k + 2, nj)
            else:
                @pl.when(q < (N_CHUNKS // RING) - 1)
                def _():
                    start_in(k + 2, nj)
            pltpu.sync_copy(slot(j), acc.at[idx2d.at[k]], add=True)
        return carry
    lax.fori_loop(0, N_CHUNKS // RING, p1body, 0)
    start_in(0, 0)
    start_in(1, 1)
    plsc.subcore_barrier()

    pltpu.sync_copy(acc, pooled)
    pltpu.sync_copy(noise_hbm.at[pl.ds(0, N_SEG), pl.ds(ch0, CH_HALF)], noise_v)

    def p2body(seg, carry):
        for v in range(N_VECS):
            p = pooled[seg, pl.ds(v * LANES, LANES)]
            nz = noise_v[seg, pl.ds(v * LANES, LANES)]
            inv_v[seg, pl.ds(v * LANES, LANES)] = 1.0 / ((0.875 + 0.25 * nz) * p)
        return carry
    lax.fori_loop(0, N_SEG, p2body, 0)

    def compute_chunk(k, j, o):
        base_j = j * CHUNK
        base_o = o * CHUNK

        def gbody(g, gcarry):
            segvec = segvec_of(k, g)
            s0 = segvec[0]
            base = base_j + g * LANES
            baseo = base_o + g * LANES

            def fast():
                ivs = [inv_v[s0, pl.ds(v * LANES, LANES)]
                       for v in range(N_VECS)]

                @plsc.parallel_loop(0, LANES, unroll=2)
                def mbody(r):
                    for v in range(N_VECS):
                        col = v * LANES
                        obuf[baseo + r, pl.ds(col, LANES)] = (
                            ring[base + r, pl.ds(col, LANES)] * ivs[v])

            def slow():
                for i in range(LANES):
                    seg = segvec[i]

                    def vbody(v, vcarry, _i=i, _seg=seg):
                        col = v * LANES
                        obuf[baseo + _i, pl.ds(col, LANES)] = (
                            ring[base + _i, pl.ds(col, LANES)]
                            * inv_v[_seg, pl.ds(col, LANES)])
                        return vcarry
                    lax.fori_loop(0, N_VECS, vbody, 0)

            lax.cond(s0 == segvec[LANES - 1], fast, slow)
            return gcarry
        lax.fori_loop(0, CHUNK // LANES, gbody, 0)

    def p3body(q, carry):
        for j in range(RING):
            k = q * RING + j
            o = j % 2
            wait_in(j)
            if j < 2:
                @pl.when(q > 0)
                def _():
                    wait_out(o)
            else:
                wait_out(o)
            compute_chunk(k, j, o)
            start_out(k, o)
            nj = (j + 2) % RING
            if j < 2:
                start_in(k + 2, nj)
            else:
                @pl.when(q < (N_CHUNKS // RING) - 1)
                def _():
                    start_in(k + 2, nj)
        return carry
    lax.fori_loop(0, N_CHUNKS // RING, p3body, 0)
    wait_out(0)
    wait_out(1)


def kernel(features, segment_ids, rand_noise):
    seg2d = segment_ids.astype(jnp.int32).reshape(N_ROWS // CHUNK, CHUNK)
    mesh = plsc.VectorSubcoreMesh(core_axis_name="c", subcore_axis_name="s")
    run = pl.kernel(
        _rescale_body,
        mesh=mesh,
        out_type=jax.ShapeDtypeStruct((N_ROWS, N_CH), jnp.float32),
        scratch_types=[
            pltpu.VMEM((RING * CHUNK, CH_HALF), jnp.float32),
            pltpu.VMEM((2 * CHUNK, CH_HALF), jnp.float32),
            pltpu.VMEM((ROWS_PER_TILE // CHUNK, CHUNK), jnp.int32),
            pltpu.VMEM((N_SEG, CH_HALF), jnp.float32),
            pltpu.VMEM((N_SEG, CH_HALF), jnp.float32),
            pltpu.VMEM((N_SEG, CH_HALF), jnp.float32),
            pltpu.VMEM_SHARED((N_SEG, CH_HALF), jnp.float32),
        ] + [pltpu.SemaphoreType.DMA] * 8,
    )
    return run(features, seg2d, rand_noise)


# speedup vs baseline: 1.2621x; 1.0183x over previous
import jax
import jax.numpy as jnp
from jax import lax
from jax.experimental import pallas as pl
from jax.experimental.pallas import tpu as pltpu
from jax.experimental.pallas import tpu_sc as plsc

N_ROWS = 32768
N_CH = 256
N_SEG = 16
N_CORES = 2
N_SUBCORES = 16
LANES = 16

CH_HALF = N_CH // N_CORES
ROWS_PER_TILE = N_ROWS // N_SUBCORES
CHUNK = 128
N_CHUNKS = ROWS_PER_TILE // CHUNK
RING = 4
N_VECS = CH_HALF // LANES


def _rescale_body(feat_hbm, seg2d_hbm, noise_hbm, out_hbm,
                  ring, idx2d, pooled, noise_v, inv_v, acc,
                  in0, in1, in2, in3, ot0, ot1, ot2, ot3):
    in_sems = (in0, in1, in2, in3)
    out_sems = (ot0, ot1, ot2, ot3)
    c = lax.axis_index("c")
    s = lax.axis_index("s")
    ch0 = c * CH_HALF
    row0 = s * ROWS_PER_TILE

    def feat_src(k):
        return feat_hbm.at[pl.ds(row0 + k * CHUNK, CHUNK), pl.ds(ch0, CH_HALF)]

    def out_dst(k):
        return out_hbm.at[pl.ds(row0 + k * CHUNK, CHUNK), pl.ds(ch0, CH_HALF)]

    def slot(j):
        return ring.at[pl.ds(j * CHUNK, CHUNK)]

    def start_in(k, j):
        pltpu.async_copy(feat_src(k), slot(j), in_sems[j])

    def wait_in(j):
        pltpu.make_async_copy(feat_src(0), slot(j), in_sems[j]).wait()

    def start_out(k, j):
        pltpu.async_copy(slot(j), out_dst(k), out_sems[j])

    def wait_out(j):
        pltpu.make_async_copy(slot(j), out_dst(0), out_sems[j]).wait()

    def segvec_of(k, g):
        gi = k * CHUNK + g * LANES
        return idx2d[lax.shift_right_logical(gi, 7),
                     pl.ds(lax.bitwise_and(gi, 127), LANES)]

    start_in(0, 0)
    start_in(1, 1)

    pltpu.sync_copy(
        seg2d_hbm.at[pl.ds(s * (ROWS_PER_TILE // CHUNK), ROWS_PER_TILE // CHUNK)],
        idx2d)

    @pl.when(s == 0)
    def _():
        def zbody(seg, carry):
            for v in range(N_VECS):
                pooled[seg, pl.ds(v * LANES, LANES)] = jnp.zeros(
                    (LANES,), jnp.float32)
            return carry
        lax.fori_loop(0, N_SEG, zbody, 0)
        pltpu.sync_copy(pooled, acc)
    plsc.subcore_barrier()

    def p1body(q, carry):
        for j in range(RING):
            k = q * RING + j
            wait_in(j)
            nj = (j + 2) % RING
            if j < 2:
                start_in(k + 2, nj)
            else:
                @pl.when(q < (N_CHUNKS // RING) - 1)
                def _():
                    start_in(k + 2, nj)
            pltpu.sync_copy(slot(j), acc.at[idx2d.at[k]], add=True)
        return carry
    lax.fori_loop(0, N_CHUNKS // RING, p1body, 0)
    start_in(0, 0)
    start_in(1, 1)
    plsc.subcore_barrier()

    pltpu.sync_copy(acc, pooled)
    pltpu.sync_copy(noise_hbm.at[pl.ds(0, N_SEG), pl.ds(ch0, CH_HALF)], noise_v)

    def p2body(seg, carry):
        for v in range(N_VECS):
            p = pooled[seg, pl.ds(v * LANES, LANES)]
            nz = noise_v[seg, pl.ds(v * LANES, LANES)]
            inv_v[seg, pl.ds(v * LANES, LANES)] = 1.0 / ((0.875 + 0.25 * nz) * p)
        return carry
    lax.fori_loop(0, N_SEG, p2body, 0)

    def compute_chunk(k, j):
        base_j = j * CHUNK

        def gbody(g, gcarry):
            segvec = segvec_of(k, g)
            s0 = segvec[0]
            base = base_j + g * LANES

            def fast():
                ivs = [inv_v[s0, pl.ds(v * LANES, LANES)]
                       for v in range(N_VECS)]

                @plsc.parallel_loop(0, LANES, unroll=2)
                def mbody(r):
                    for v in range(N_VECS):
                        col = v * LANES
                        ring[base + r, pl.ds(col, LANES)] = (
                            ring[base + r, pl.ds(col, LANES)] * ivs[v])

            def slow():
                for i in range(LANES):
                    seg = segvec[i]

                    def vbody(v, vcarry, _i=i, _seg=seg):
                        col = v * LANES
                        ring[base + _i, pl.ds(col, LANES)] = (
                            ring[base + _i, pl.ds(col, LANES)]
                            * inv_v[_seg, pl.ds(col, LANES)])
                        return vcarry
                    lax.fori_loop(0, N_VECS, vbody, 0)

            lax.cond(s0 == segvec[LANES - 1], fast, slow)
            return gcarry
        lax.fori_loop(0, CHUNK // LANES, gbody, 0)

    def p3body(q, carry):
        for j in range(RING):
            k = q * RING + j
            wait_in(j)
            compute_chunk(k, j)
            start_out(k, j)
            nj = (j + 2) % RING
            if j < 2:
                @pl.when(q > 0)
                def _():
                    wait_out(nj)
                start_in(k + 2, nj)
            else:
                @pl.when(q < (N_CHUNKS // RING) - 1)
                def _():
                    wait_out(nj)
                    start_in(k + 2, nj)
        return carry
    lax.fori_loop(0, N_CHUNKS // RING, p3body, 0)
    wait_out(0)
    wait_out(1)
    wait_out(2)
    wait_out(3)


def kernel(features, segment_ids, rand_noise):
    seg2d = segment_ids.astype(jnp.int32).reshape(N_ROWS // CHUNK, CHUNK)
    mesh = plsc.VectorSubcoreMesh(core_axis_name="c", subcore_axis_name="s")
    run = pl.kernel(
        _rescale_body,
        mesh=mesh,
        out_type=jax.ShapeDtypeStruct((N_ROWS, N_CH), jnp.float32),
        scratch_types=[
            pltpu.VMEM((RING * CHUNK, CH_HALF), jnp.float32),
            pltpu.VMEM((ROWS_PER_TILE // CHUNK, CHUNK), jnp.int32),
            pltpu.VMEM((N_SEG, CH_HALF), jnp.float32),
            pltpu.VMEM((N_SEG, CH_HALF), jnp.float32),
            pltpu.VMEM((N_SEG, CH_HALF), jnp.float32),
            pltpu.VMEM_SHARED((N_SEG, CH_HALF), jnp.float32),
        ] + [pltpu.SemaphoreType.DMA] * 8,
    )
    return run(features, seg2d, rand_noise)
